# baseline (speedup 1.0000x reference)
import functools

import jax
import jax.numpy as jnp
from jax import lax
from jax.experimental import pallas as pl
from jax.experimental.pallas import tpu as pltpu
from jax.experimental.pallas import tpu_sc as plsc

VOCAB = 1000000
D = 32
BATCH = 16384
FIELDS = 26

NC = 2
NS = 16
NW = NC * NS

B_PER_W = BATCH // NW * FIELDS
BT_PER_W = BATCH // NW // 128
ATOMS = FIELDS * BT_PER_W
TPAD = 129
NB = 4


def _emb_body(x_hbm, lut_hbm, out_hbm, xbuf, ibuf, gbs, tbs, gss, tss):
    cid = lax.axis_index("c")
    sid = lax.axis_index("s")
    wid = sid * NC + cid
    pltpu.sync_copy(x_hbm.at[pl.ds(wid * B_PER_W, B_PER_W)], xbuf)

    iota = lax.iota(jnp.int32, 16)
    v26 = iota * FIELDS

    for a in range(ATOMS):
        f, j = a // BT_PER_W, a % BT_PER_W
        for g in range(8):
            src = v26 + ((128 * j + 16 * g) * FIELDS + f)
            ibuf[pl.ds(a * 128 + g * 16, 16)] = plsc.load_gather(xbuf, [src])

    def gather(a, k):
        pltpu.make_async_copy(
            lut_hbm.at[ibuf.at[pl.ds(a * 128, 128)]], gbs[k], gss[k]
        ).start()

    def drain_out(k):
        for _ in range(4):
            pltpu.make_async_copy(tbs[k].at[pl.ds(0, 8), pl.ds(0, 128)],
                                  out_hbm.at[0, 0, 0], tss[k]).wait()

    def atom(a, k, not_first, has_next):
        @pl.when(not_first)
        def _():
            drain_out(k)

        pltpu.make_async_copy(
            lut_hbm.at[ibuf.at[pl.ds(a * 128, 128)]], gbs[k], gss[k]
        ).wait()

        gb, tb = gbs[k], tbs[k]
        for bl in range(128):
            blv = jnp.full((16,), bl, jnp.int32)
            r0 = plsc.load_gather(gb, [blv, iota])
            r1 = plsc.load_gather(gb, [blv, iota + 16])
            plsc.store_scatter(tb, [iota, blv], r0)
            plsc.store_scatter(tb, [iota + 16, blv], r1)

        f = a // BT_PER_W
        bt = wid * BT_PER_W + (a - f * BT_PER_W)
        for dt in range(4):
            pltpu.make_async_copy(
                tb.at[pl.ds(dt * 8, 8), pl.ds(0, 128)],
                out_hbm.at[f, dt, bt], tss[k],
            ).start()

        @pl.when(has_next)
        def _():
            gather(a + NB, k)

    for k in range(NB):
        gather(k, k)

    def body(i, _):
        a0 = NB * i
        for k in range(NB):
            atom(a0 + k, k, a0 + k >= NB, a0 + k + NB < ATOMS)
        return 0

    lax.fori_loop(0, ATOMS // NB, body, 0)

    for k in range(NB):
        drain_out(k)


_emb = functools.partial(
    pl.kernel,
    out_type=jax.ShapeDtypeStruct((FIELDS, 4, 128, 8, 128), jnp.float32),
    mesh=plsc.VectorSubcoreMesh(core_axis_name="c", subcore_axis_name="s"),
    scratch_types=[
        pltpu.VMEM((B_PER_W,), jnp.int32),
        pltpu.VMEM((B_PER_W,), jnp.int32),
        [pltpu.VMEM((128, D), jnp.float32) for _ in range(NB)],
        [pltpu.VMEM((D, TPAD), jnp.float32) for _ in range(NB)],
        [pltpu.SemaphoreType.DMA for _ in range(NB)],
        [pltpu.SemaphoreType.DMA for _ in range(NB)],
    ],
    compiler_params=pltpu.CompilerParams(
        use_tc_tiling_on_sc=False, needs_layout_passes=False
    ),
)(_emb_body)


@jax.jit
def kernel(x, lut):
    xi = (x.astype(jnp.int32) * 4).reshape(BATCH * FIELDS)
    lut_p = jnp.pad(lut, ((0, 0), (0, 3 * D)))
    out5 = _emb(xi, lut_p.reshape(4 * VOCAB, D))
    return out5.transpose(2, 4, 0, 1, 3).reshape(BATCH, FIELDS, D)
